# Initial kernel scaffold; baseline (speedup 1.0000x reference)
import jax
import jax.numpy as jnp
from jax.experimental import pallas as pl


def kernel(X, A, W, W0, b0, gamma0, beta0, W1, b1):
    raise NotImplementedError("write your pallas kernel here")


# speedup vs baseline: 11.5067x; 11.5067x over previous
import functools

import numpy as np

import jax
import jax.numpy as jnp
from jax import lax
from jax.experimental import pallas as pl
from jax.experimental.pallas import tpu as pltpu
from jax.experimental.pallas import tpu_sc as plsc

N = 10000
E = 320000
D = 128

NC = 2
NS = 16
L = 16
NW = NC * NS

N_PAD = 10240
ROWS_PER_TILE = N_PAD // NS
CHUNK = 128
PER_W = 10112
E_PAD = PER_W * NW
N_CHUNKS = PER_W // CHUNK

_mesh = plsc.VectorSubcoreMesh(core_axis_name="c", subcore_axis_name="s")


@functools.partial(
    pl.kernel,
    out_type=jax.ShapeDtypeStruct((NC, N_PAD), jnp.float32),
    mesh=_mesh,
    scratch_types=[
        pltpu.VMEM((CHUNK,), jnp.int32),
        pltpu.VMEM((CHUNK,), jnp.float32),
        pltpu.VMEM((ROWS_PER_TILE,), jnp.float32),
        pltpu.VMEM_SHARED((N_PAD,), jnp.float32),
    ],
)
def _deg_kernel(dst_hbm, w_hbm, out_hbm, idx_v, w_v, zbuf, deg_sh):
    c = lax.axis_index("c")
    s = lax.axis_index("s")
    g = c * NS + s

    def _z(i, _):
        zbuf[pl.ds(i * L, L)] = jnp.zeros((L,), jnp.float32)
        return 0
    lax.fori_loop(0, ROWS_PER_TILE // L, _z, 0)
    pltpu.sync_copy(zbuf, deg_sh.at[pl.ds(s * ROWS_PER_TILE, ROWS_PER_TILE)])
    plsc.subcore_barrier()

    def _chunk(j, _):
        base = g * PER_W + j * CHUNK
        pltpu.sync_copy(dst_hbm.at[pl.ds(base, CHUNK)], idx_v)
        pltpu.sync_copy(w_hbm.at[pl.ds(base, CHUNK)], w_v)
        pltpu.sync_copy(w_v, deg_sh.at[idx_v], add=True)
        return 0
    lax.fori_loop(0, N_CHUNKS, _chunk, 0)
    plsc.subcore_barrier()

    pltpu.sync_copy(deg_sh.at[pl.ds(s * ROWS_PER_TILE, ROWS_PER_TILE)],
                    out_hbm.at[c, pl.ds(s * ROWS_PER_TILE, ROWS_PER_TILE)])


@functools.partial(
    pl.kernel,
    out_type=jax.ShapeDtypeStruct((NC, N_PAD, D), jnp.float32),
    mesh=_mesh,
    scratch_types=[
        pltpu.VMEM((CHUNK,), jnp.int32),
        pltpu.VMEM((CHUNK,), jnp.int32),
        pltpu.VMEM((CHUNK,), jnp.float32),
        pltpu.VMEM((CHUNK, D), jnp.float32),
        pltpu.VMEM_SHARED((N_PAD, D), jnp.float32),
        pltpu.SemaphoreType.DMA,
    ],
    compiler_params=pltpu.CompilerParams(needs_layout_passes=False),
)
def _agg_kernel(y_hbm, src_hbm, dst_hbm, w_hbm, out_hbm,
                idx_s, idx_d, w_v, rows, acc_sh, sem):
    c = lax.axis_index("c")
    s = lax.axis_index("s")
    g = c * NS + s

    def _zr(r, _):
        for col in range(D // L):
            rows[r, pl.ds(col * L, L)] = jnp.zeros((L,), jnp.float32)
        return 0
    lax.fori_loop(0, CHUNK, _zr, 0)
    for k in range(ROWS_PER_TILE // CHUNK):
        pltpu.sync_copy(
            rows, acc_sh.at[pl.ds(s * ROWS_PER_TILE + k * CHUNK, CHUNK)])
    plsc.subcore_barrier()

    def _chunk(j, _):
        base = g * PER_W + j * CHUNK
        pltpu.sync_copy(src_hbm.at[pl.ds(base, CHUNK)], idx_s)
        pltpu.sync_copy(dst_hbm.at[pl.ds(base, CHUNK)], idx_d)
        pltpu.sync_copy(w_hbm.at[pl.ds(base, CHUNK)], w_v)
        pltpu.async_copy(y_hbm.at[idx_s], rows, sem).wait()

        def _scale(r, _):
            wspl = plsc.load_gather(w_v, [jnp.full((L,), r, jnp.int32)])
            for col in range(D // L):
                sl = pl.ds(col * L, L)
                rows[r, sl] = rows[r, sl] * wspl
            return 0
        lax.fori_loop(0, CHUNK, _scale, 0)

        pltpu.sync_copy(rows, acc_sh.at[idx_d], add=True)
        return 0
    lax.fori_loop(0, N_CHUNKS, _chunk, 0)
    plsc.subcore_barrier()

    pltpu.sync_copy(acc_sh.at[pl.ds(s * ROWS_PER_TILE, ROWS_PER_TILE)],
                    out_hbm.at[c, pl.ds(s * ROWS_PER_TILE, ROWS_PER_TILE)])


def _mm_body(x_ref, w0_ref, degp_ref, y_ref):
    deg = degp_ref[:, 0] + degp_ref[:, 1] + 1.0
    dinv = jnp.where(deg > 0, lax.rsqrt(deg), 0.0)
    xw = lax.dot_general(x_ref[...], w0_ref[...], (((1,), (1,)), ((), ())),
                         preferred_element_type=jnp.float32)
    y_ref[...] = dinv[:, None] * xw


_MM_BLK = 1000


def _matmul_y(X, W0, degp_t):
    grid = N // _MM_BLK
    return pl.pallas_call(
        _mm_body,
        grid=(grid,),
        in_specs=[
            pl.BlockSpec((_MM_BLK, D), lambda i: (i, 0)),
            pl.BlockSpec((D, D), lambda i: (0, 0)),
            pl.BlockSpec((_MM_BLK, NC), lambda i: (i, 0)),
        ],
        out_specs=pl.BlockSpec((_MM_BLK, D), lambda i: (i, 0)),
        out_shape=jax.ShapeDtypeStruct((N, D), jnp.float32),
    )(X, W0, degp_t)


def _final_body(accp_ref, y_ref, degp_ref, b0_ref, g0_ref, be0_ref,
                w1_ref, b1_ref, out_ref):
    deg = degp_ref[0, :N] + degp_ref[1, :N] + 1.0
    dinv = jnp.where(deg > 0, lax.rsqrt(deg), 0.0)
    acc = accp_ref[0, :N, :] + accp_ref[1, :N, :]
    h = dinv[:, None] * (acc + y_ref[...]) + b0_ref[...]
    mean = jnp.mean(h, axis=0)
    var = jnp.mean((h - mean) ** 2, axis=0)
    h = (h - mean) / jnp.sqrt(var + 1e-5) * g0_ref[...] + be0_ref[...]
    h = jnp.where(h >= 0, h, 0.01 * h)
    out_ref[...] = lax.dot_general(
        h, w1_ref[...], (((1,), (1,)), ((), ())),
        preferred_element_type=jnp.float32) + b1_ref[...]


def _final(accp, y, degp, b0, gamma0, beta0, W1, b1):
    return pl.pallas_call(
        _final_body,
        out_shape=jax.ShapeDtypeStruct((N, D), jnp.float32),
    )(accp, y, degp, b0, gamma0, beta0, W1, b1)


def kernel(X, A, W, W0, b0, gamma0, beta0, W1, b1):
    pad = E_PAD - E
    src = jnp.concatenate([A[0], jnp.zeros((pad,), A.dtype)])
    dst = jnp.concatenate([A[1], jnp.zeros((pad,), A.dtype)])
    w = jnp.concatenate([W, jnp.zeros((pad,), W.dtype)])

    degp = _deg_kernel(dst, w)
    y = _matmul_y(X, W0, degp.T)
    accp = _agg_kernel(y, src, dst, w)
    return _final(accp, y, degp, b0, gamma0, beta0, W1, b1)


# speedup vs baseline: 14.4253x; 1.2537x over previous
import functools

import jax
import jax.numpy as jnp
from jax import lax
from jax.experimental import pallas as pl
from jax.experimental.pallas import tpu as pltpu
from jax.experimental.pallas import tpu_sc as plsc

N = 10000
E = 320000
D = 128

NC = 2
NS = 16
L = 16
NW = NC * NS

N_PAD = 10240
ROWS_PER_TILE = N_PAD // NS
CHUNK = 128
PER_W = 10240
E_PAD = PER_W * NW
N_CHUNKS = PER_W // CHUNK
HALF_CH = N_CHUNKS // 2
NCH_TOT = E_PAD // CHUNK

_mesh = plsc.VectorSubcoreMesh(core_axis_name="c", subcore_axis_name="s")


@functools.partial(
    pl.kernel,
    out_type=jax.ShapeDtypeStruct((NC, N_PAD), jnp.float32),
    mesh=_mesh,
    scratch_types=[
        pltpu.VMEM((N_CHUNKS, CHUNK), jnp.int32),
        pltpu.VMEM((N_CHUNKS, CHUNK), jnp.float32),
        pltpu.VMEM((ROWS_PER_TILE,), jnp.float32),
        pltpu.VMEM_SHARED((N_PAD,), jnp.float32),
        pltpu.SemaphoreType.DMA,
    ],
)
def _deg_kernel(dst_hbm, w_hbm, out_hbm, idx_all, w_all, zbuf, deg_sh, sem):
    c = lax.axis_index("c")
    s = lax.axis_index("s")
    g = c * NS + s

    def _z(i, _):
        zbuf[pl.ds(i * L, L)] = jnp.zeros((L,), jnp.float32)
        return 0
    lax.fori_loop(0, ROWS_PER_TILE // L, _z, 0)
    pltpu.sync_copy(zbuf, deg_sh.at[pl.ds(s * ROWS_PER_TILE, ROWS_PER_TILE)])
    plsc.subcore_barrier()

    row0 = g * N_CHUNKS
    pltpu.sync_copy(dst_hbm.at[pl.ds(row0, N_CHUNKS)], idx_all)
    pltpu.sync_copy(w_hbm.at[pl.ds(row0, N_CHUNKS)], w_all)

    K = 16
    for j0 in range(0, N_CHUNKS, K):
        jn = min(j0 + K, N_CHUNKS)
        for j in range(j0, jn):
            pltpu.async_copy(w_all.at[j], deg_sh.at[idx_all.at[j]], sem,
                             add=True)
        for j in range(j0, jn):
            pltpu.make_async_copy(w_all.at[j], deg_sh.at[idx_all.at[j]],
                                  sem).wait()
    plsc.subcore_barrier()

    pltpu.sync_copy(deg_sh.at[pl.ds(s * ROWS_PER_TILE, ROWS_PER_TILE)],
                    out_hbm.at[c, pl.ds(s * ROWS_PER_TILE, ROWS_PER_TILE)])


@functools.partial(
    pl.kernel,
    out_type=jax.ShapeDtypeStruct((NC, N_PAD, D), jnp.float32),
    mesh=_mesh,
    scratch_types=[
        pltpu.VMEM((HALF_CH, CHUNK), jnp.int32),
        pltpu.VMEM((HALF_CH, CHUNK), jnp.int32),
        pltpu.VMEM((HALF_CH, CHUNK), jnp.float32),
        pltpu.VMEM((2, CHUNK, D), jnp.float32),
        pltpu.VMEM_SHARED((N_PAD, D), jnp.float32),
        pltpu.SemaphoreType.DMA,
        pltpu.SemaphoreType.DMA,
    ],
    compiler_params=pltpu.CompilerParams(needs_layout_passes=False),
)
def _agg_kernel(y_hbm, src_hbm, dst_hbm, w_hbm, out_hbm,
                idx_s, idx_d, w_all, rows2, acc_sh, sem_g, sem_s):
    c = lax.axis_index("c")
    s = lax.axis_index("s")
    g = c * NS + s

    def _zr(r, _):
        for col in range(D // L):
            rows2[0, r, pl.ds(col * L, L)] = jnp.zeros((L,), jnp.float32)
        return 0
    lax.fori_loop(0, CHUNK, _zr, 0)
    for k in range(ROWS_PER_TILE // CHUNK):
        pltpu.sync_copy(
            rows2.at[0], acc_sh.at[pl.ds(s * ROWS_PER_TILE + k * CHUNK,
                                         CHUNK)])
    plsc.subcore_barrier()

    for half in range(2):
        row0 = g * N_CHUNKS + half * HALF_CH
        pltpu.sync_copy(src_hbm.at[pl.ds(row0, HALF_CH)], idx_s)
        pltpu.sync_copy(dst_hbm.at[pl.ds(row0, HALF_CH)], idx_d)
        pltpu.sync_copy(w_hbm.at[pl.ds(row0, HALF_CH)], w_all)

        pltpu.async_copy(y_hbm.at[idx_s.at[0]], rows2.at[0], sem_g)

        def _chunk(j, _):
            p = lax.rem(j, 2)
            q = 1 - p
            pltpu.make_async_copy(y_hbm.at[idx_s.at[j]], rows2.at[p],
                                  sem_g).wait()

            @pl.when(j >= 1)
            def _():
                pltpu.make_async_copy(rows2.at[q],
                                      acc_sh.at[idx_d.at[j - 1]],
                                      sem_s).wait()

            @pl.when(j <= HALF_CH - 2)
            def _():
                pltpu.async_copy(y_hbm.at[idx_s.at[j + 1]], rows2.at[q],
                                 sem_g)

            def _scale(r, _):
                wspl = plsc.load_gather(
                    w_all, [jnp.full((L,), j, jnp.int32),
                            jnp.full((L,), r, jnp.int32)])
                for col in range(D // L):
                    sl = pl.ds(col * L, L)
                    rows2[p, r, sl] = rows2[p, r, sl] * wspl
                return 0
            lax.fori_loop(0, CHUNK, _scale, 0)

            pltpu.async_copy(rows2.at[p], acc_sh.at[idx_d.at[j]], sem_s,
                             add=True)
            return 0
        lax.fori_loop(0, HALF_CH, _chunk, 0)
        pltpu.make_async_copy(rows2.at[(HALF_CH - 1) % 2],
                              acc_sh.at[idx_d.at[HALF_CH - 1]],
                              sem_s).wait()
    plsc.subcore_barrier()

    pltpu.sync_copy(acc_sh.at[pl.ds(s * ROWS_PER_TILE, ROWS_PER_TILE)],
                    out_hbm.at[c, pl.ds(s * ROWS_PER_TILE, ROWS_PER_TILE)])


def _mm_body(x_ref, w0_ref, degp_ref, y_ref):
    deg = degp_ref[:, 0] + degp_ref[:, 1] + 1.0
    dinv = jnp.where(deg > 0, lax.rsqrt(deg), 0.0)
    xw = lax.dot_general(x_ref[...], w0_ref[...], (((1,), (1,)), ((), ())),
                         preferred_element_type=jnp.float32)
    y_ref[...] = dinv[:, None] * xw


_MM_BLK = 1000


def _matmul_y(X, W0, degp_t):
    grid = N // _MM_BLK
    return pl.pallas_call(
        _mm_body,
        grid=(grid,),
        in_specs=[
            pl.BlockSpec((_MM_BLK, D), lambda i: (i, 0)),
            pl.BlockSpec((D, D), lambda i: (0, 0)),
            pl.BlockSpec((_MM_BLK, NC), lambda i: (i, 0)),
        ],
        out_specs=pl.BlockSpec((_MM_BLK, D), lambda i: (i, 0)),
        out_shape=jax.ShapeDtypeStruct((N, D), jnp.float32),
    )(X, W0, degp_t)


def _final_body(accp_ref, y_ref, degp_ref, b0_ref, g0_ref, be0_ref,
                w1_ref, b1_ref, out_ref):
    deg = degp_ref[0, :N] + degp_ref[1, :N] + 1.0
    dinv = jnp.where(deg > 0, lax.rsqrt(deg), 0.0)
    acc = accp_ref[0, :N, :] + accp_ref[1, :N, :]
    h = dinv[:, None] * (acc + y_ref[...]) + b0_ref[...]
    mean = jnp.mean(h, axis=0)
    var = jnp.mean((h - mean) ** 2, axis=0)
    h = (h - mean) / jnp.sqrt(var + 1e-5) * g0_ref[...] + be0_ref[...]
    h = jnp.where(h >= 0, h, 0.01 * h)
    out_ref[...] = lax.dot_general(
        h, w1_ref[...], (((1,), (1,)), ((), ())),
        preferred_element_type=jnp.float32) + b1_ref[...]


def _final(accp, y, degp, b0, gamma0, beta0, W1, b1):
    return pl.pallas_call(
        _final_body,
        out_shape=jax.ShapeDtypeStruct((N, D), jnp.float32),
    )(accp, y, degp, b0, gamma0, beta0, W1, b1)


def kernel(X, A, W, W0, b0, gamma0, beta0, W1, b1):
    pad = E_PAD - E
    src = jnp.concatenate([A[0], jnp.zeros((pad,), A.dtype)])
    dst = jnp.concatenate([A[1], jnp.zeros((pad,), A.dtype)])
    w = jnp.concatenate([W, jnp.zeros((pad,), W.dtype)])
    src2 = src.reshape(NCH_TOT, CHUNK)
    dst2 = dst.reshape(NCH_TOT, CHUNK)
    w2 = w.reshape(NCH_TOT, CHUNK)

    degp = _deg_kernel(dst2, w2)
    y = _matmul_y(X, W0, degp.T)
    accp = _agg_kernel(y, src2, dst2, w2)
    return _final(accp, y, degp, b0, gamma0, beta0, W1, b1)


# speedup vs baseline: 14.6252x; 1.0139x over previous
import functools

import jax
import jax.numpy as jnp
from jax import lax
from jax.experimental import pallas as pl
from jax.experimental.pallas import tpu as pltpu
from jax.experimental.pallas import tpu_sc as plsc

N = 10000
E = 320000
D = 128

NC = 2
NS = 16
L = 16
NW = NC * NS

N_PAD = 10240
ROWS_PER_TILE = N_PAD // NS
CHUNK = 128
PER_W = 10240
E_PAD = PER_W * NW
N_CHUNKS = PER_W // CHUNK
HALF_CH = N_CHUNKS // 2
NCH_TOT = E_PAD // CHUNK

_mesh = plsc.VectorSubcoreMesh(core_axis_name="c", subcore_axis_name="s")


@functools.partial(
    pl.kernel,
    out_type=jax.ShapeDtypeStruct((NC, N_PAD), jnp.float32),
    mesh=_mesh,
    scratch_types=[
        pltpu.VMEM((N_CHUNKS, CHUNK), jnp.int32),
        pltpu.VMEM((N_CHUNKS, CHUNK), jnp.float32),
        pltpu.VMEM((ROWS_PER_TILE,), jnp.float32),
        pltpu.VMEM_SHARED((N_PAD,), jnp.float32),
        pltpu.SemaphoreType.DMA,
    ],
)
def _deg_kernel(dst_hbm, w_hbm, out_hbm, idx_all, w_all, zbuf, deg_sh, sem):
    c = lax.axis_index("c")
    s = lax.axis_index("s")
    g = c * NS + s

    def _z(i, _):
        zbuf[pl.ds(i * L, L)] = jnp.zeros((L,), jnp.float32)
        return 0
    lax.fori_loop(0, ROWS_PER_TILE // L, _z, 0)
    pltpu.sync_copy(zbuf, deg_sh.at[pl.ds(s * ROWS_PER_TILE, ROWS_PER_TILE)])
    plsc.subcore_barrier()

    row0 = g * N_CHUNKS
    pltpu.sync_copy(dst_hbm.at[pl.ds(row0, N_CHUNKS)], idx_all)
    pltpu.sync_copy(w_hbm.at[pl.ds(row0, N_CHUNKS)], w_all)

    K = 16
    for j0 in range(0, N_CHUNKS, K):
        jn = min(j0 + K, N_CHUNKS)
        for j in range(j0, jn):
            pltpu.async_copy(w_all.at[j], deg_sh.at[idx_all.at[j]], sem,
                             add=True)
        for j in range(j0, jn):
            pltpu.make_async_copy(w_all.at[j], deg_sh.at[idx_all.at[j]],
                                  sem).wait()
    plsc.subcore_barrier()

    pltpu.sync_copy(deg_sh.at[pl.ds(s * ROWS_PER_TILE, ROWS_PER_TILE)],
                    out_hbm.at[c, pl.ds(s * ROWS_PER_TILE, ROWS_PER_TILE)])


@functools.partial(
    pl.kernel,
    out_type=jax.ShapeDtypeStruct((NC, N_PAD, D), jnp.float32),
    mesh=_mesh,
    scratch_types=[
        pltpu.VMEM((HALF_CH, CHUNK), jnp.int32),
        pltpu.VMEM((HALF_CH, CHUNK), jnp.int32),
        pltpu.VMEM((HALF_CH, CHUNK), jnp.float32),
        pltpu.VMEM((2, CHUNK, D), jnp.float32),
        pltpu.VMEM_SHARED((N_PAD, D), jnp.float32),
        pltpu.SemaphoreType.DMA,
        pltpu.SemaphoreType.DMA,
    ],
    compiler_params=pltpu.CompilerParams(needs_layout_passes=False),
)
def _agg_kernel(y_hbm, src_hbm, dst_hbm, w_hbm, out_hbm,
                idx_s, idx_d, w_all, rows2, acc_sh, sem_g, sem_s):
    c = lax.axis_index("c")
    s = lax.axis_index("s")
    g = c * NS + s

    def _zr(r, _):
        for col in range(D // L):
            rows2[0, r, pl.ds(col * L, L)] = jnp.zeros((L,), jnp.float32)
        return 0
    lax.fori_loop(0, CHUNK, _zr, 0)
    for k in range(ROWS_PER_TILE // CHUNK):
        pltpu.sync_copy(
            rows2.at[0], acc_sh.at[pl.ds(s * ROWS_PER_TILE + k * CHUNK,
                                         CHUNK)])
    plsc.subcore_barrier()

    for half in range(2):
        row0 = g * N_CHUNKS + half * HALF_CH
        pltpu.sync_copy(src_hbm.at[pl.ds(row0, HALF_CH)], idx_s)
        pltpu.sync_copy(dst_hbm.at[pl.ds(row0, HALF_CH)], idx_d)
        pltpu.sync_copy(w_hbm.at[pl.ds(row0, HALF_CH)], w_all)

        pltpu.async_copy(y_hbm.at[idx_s.at[0]], rows2.at[0], sem_g)

        def _chunk(j, _):
            p = lax.rem(j, 2)
            q = 1 - p
            pltpu.make_async_copy(y_hbm.at[idx_s.at[j]], rows2.at[p],
                                  sem_g).wait()

            @pl.when(j >= 1)
            def _():
                pltpu.make_async_copy(rows2.at[q],
                                      acc_sh.at[idx_d.at[j - 1]],
                                      sem_s).wait()

            @pl.when(j <= HALF_CH - 2)
            def _():
                pltpu.async_copy(y_hbm.at[idx_s.at[j + 1]], rows2.at[q],
                                 sem_g)

            @plsc.parallel_loop(0, CHUNK, step=1, unroll=4)
            def _scale(r):
                wspl = plsc.load_gather(
                    w_all, [jnp.full((L,), j, jnp.int32),
                            jnp.full((L,), r, jnp.int32)])
                for col in range(D // L):
                    sl = pl.ds(col * L, L)
                    rows2[p, r, sl] = rows2[p, r, sl] * wspl

            pltpu.async_copy(rows2.at[p], acc_sh.at[idx_d.at[j]], sem_s,
                             add=True)
            return 0
        lax.fori_loop(0, HALF_CH, _chunk, 0)
        pltpu.make_async_copy(rows2.at[(HALF_CH - 1) % 2],
                              acc_sh.at[idx_d.at[HALF_CH - 1]],
                              sem_s).wait()
    plsc.subcore_barrier()

    pltpu.sync_copy(acc_sh.at[pl.ds(s * ROWS_PER_TILE, ROWS_PER_TILE)],
                    out_hbm.at[c, pl.ds(s * ROWS_PER_TILE, ROWS_PER_TILE)])


def _mm_body(x_ref, w0_ref, degp_ref, y_ref):
    deg = degp_ref[:, 0] + degp_ref[:, 1] + 1.0
    dinv = jnp.where(deg > 0, lax.rsqrt(deg), 0.0)
    xw = lax.dot_general(x_ref[...], w0_ref[...], (((1,), (1,)), ((), ())),
                         preferred_element_type=jnp.float32)
    y_ref[...] = dinv[:, None] * xw


_MM_BLK = 1000


def _matmul_y(X, W0, degp_t):
    grid = N // _MM_BLK
    return pl.pallas_call(
        _mm_body,
        grid=(grid,),
        in_specs=[
            pl.BlockSpec((_MM_BLK, D), lambda i: (i, 0)),
            pl.BlockSpec((D, D), lambda i: (0, 0)),
            pl.BlockSpec((_MM_BLK, NC), lambda i: (i, 0)),
        ],
        out_specs=pl.BlockSpec((_MM_BLK, D), lambda i: (i, 0)),
        out_shape=jax.ShapeDtypeStruct((N, D), jnp.float32),
    )(X, W0, degp_t)


def _final_body(accp_ref, y_ref, degp_ref, b0_ref, g0_ref, be0_ref,
                w1_ref, b1_ref, out_ref):
    deg = degp_ref[0, :N] + degp_ref[1, :N] + 1.0
    dinv = jnp.where(deg > 0, lax.rsqrt(deg), 0.0)
    acc = accp_ref[0, :N, :] + accp_ref[1, :N, :]
    h = dinv[:, None] * (acc + y_ref[...]) + b0_ref[...]
    mean = jnp.mean(h, axis=0)
    var = jnp.mean((h - mean) ** 2, axis=0)
    h = (h - mean) / jnp.sqrt(var + 1e-5) * g0_ref[...] + be0_ref[...]
    h = jnp.where(h >= 0, h, 0.01 * h)
    out_ref[...] = lax.dot_general(
        h, w1_ref[...], (((1,), (1,)), ((), ())),
        preferred_element_type=jnp.float32) + b1_ref[...]


def _final(accp, y, degp, b0, gamma0, beta0, W1, b1):
    return pl.pallas_call(
        _final_body,
        out_shape=jax.ShapeDtypeStruct((N, D), jnp.float32),
    )(accp, y, degp, b0, gamma0, beta0, W1, b1)


def kernel(X, A, W, W0, b0, gamma0, beta0, W1, b1):
    pad = E_PAD - E
    src = jnp.concatenate([A[0], jnp.zeros((pad,), A.dtype)])
    dst = jnp.concatenate([A[1], jnp.zeros((pad,), A.dtype)])
    w = jnp.concatenate([W, jnp.zeros((pad,), W.dtype)])
    src2 = src.reshape(NCH_TOT, CHUNK)
    dst2 = dst.reshape(NCH_TOT, CHUNK)
    w2 = w.reshape(NCH_TOT, CHUNK)

    degp = _deg_kernel(dst2, w2)
    y = _matmul_y(X, W0, degp.T)
    accp = _agg_kernel(y, src2, dst2, w2)
    return _final(accp, y, degp, b0, gamma0, beta0, W1, b1)


# speedup vs baseline: 37.5678x; 2.5687x over previous
import functools

import jax
import jax.numpy as jnp
from jax import lax
from jax.experimental import pallas as pl
from jax.experimental.pallas import tpu as pltpu
from jax.experimental.pallas import tpu_sc as plsc

N = 10000
E = 320000
D = 128

NC = 2
NS = 16
L = 16
NW = NC * NS

N_PAD = 10240
ROWS_PER_TILE = N_PAD // NS
CHUNK = 128
PER_W = 10240
E_PAD = PER_W * NW
N_CHUNKS = PER_W // CHUNK
HALF_CH = N_CHUNKS // 2
NCH_TOT = E_PAD // CHUNK

_mesh = plsc.VectorSubcoreMesh(core_axis_name="c", subcore_axis_name="s")


@functools.partial(
    pl.kernel,
    out_type=jax.ShapeDtypeStruct((NC, N_PAD), jnp.float32),
    mesh=_mesh,
    scratch_types=[
        pltpu.VMEM((N_CHUNKS, CHUNK), jnp.int32),
        pltpu.VMEM((N_CHUNKS, CHUNK), jnp.float32),
        pltpu.VMEM((ROWS_PER_TILE,), jnp.float32),
        pltpu.VMEM_SHARED((N_PAD,), jnp.float32),
        pltpu.SemaphoreType.DMA,
    ],
)
def _deg_kernel(dst_hbm, w_hbm, out_hbm, idx_all, w_all, zbuf, deg_sh, sem):
    c = lax.axis_index("c")
    s = lax.axis_index("s")
    g = c * NS + s

    def _z(i, _):
        zbuf[pl.ds(i * L, L)] = jnp.zeros((L,), jnp.float32)
        return 0
    lax.fori_loop(0, ROWS_PER_TILE // L, _z, 0)
    pltpu.sync_copy(zbuf, deg_sh.at[pl.ds(s * ROWS_PER_TILE, ROWS_PER_TILE)])
    plsc.subcore_barrier()

    row0 = g * N_CHUNKS
    pltpu.sync_copy(dst_hbm.at[pl.ds(row0, N_CHUNKS)], idx_all)
    pltpu.sync_copy(w_hbm.at[pl.ds(row0, N_CHUNKS)], w_all)

    K = 16
    for j0 in range(0, N_CHUNKS, K):
        jn = min(j0 + K, N_CHUNKS)
        for j in range(j0, jn):
            pltpu.async_copy(w_all.at[j], deg_sh.at[idx_all.at[j]], sem,
                             add=True)
        for j in range(j0, jn):
            pltpu.make_async_copy(w_all.at[j], deg_sh.at[idx_all.at[j]],
                                  sem).wait()
    plsc.subcore_barrier()

    pltpu.sync_copy(deg_sh.at[pl.ds(s * ROWS_PER_TILE, ROWS_PER_TILE)],
                    out_hbm.at[c, pl.ds(s * ROWS_PER_TILE, ROWS_PER_TILE)])


@functools.partial(
    pl.kernel,
    out_type=jax.ShapeDtypeStruct((NC, N_PAD, D), jnp.float32),
    mesh=_mesh,
    scratch_types=[
        pltpu.VMEM((HALF_CH, CHUNK), jnp.int32),
        pltpu.VMEM((HALF_CH, CHUNK), jnp.int32),
        pltpu.VMEM((HALF_CH, CHUNK), jnp.float32),
        pltpu.VMEM((2, CHUNK, D), jnp.float32),
        pltpu.VMEM_SHARED((N_PAD, D), jnp.float32),
        pltpu.SemaphoreType.DMA,
        pltpu.SemaphoreType.DMA,
    ],
    compiler_params=pltpu.CompilerParams(needs_layout_passes=False),
)
def _agg_kernel(y_hbm, src_hbm, dst_hbm, w_hbm, out_hbm,
                idx_s, idx_d, w_all, rows2, acc_sh, sem_g, sem_s):
    c = lax.axis_index("c")
    s = lax.axis_index("s")
    g = c * NS + s

    def _zr(r, _):
        for col in range(D // L):
            rows2[0, r, pl.ds(col * L, L)] = jnp.zeros((L,), jnp.float32)
        return 0
    lax.fori_loop(0, CHUNK, _zr, 0)
    for k in range(ROWS_PER_TILE // CHUNK):
        pltpu.sync_copy(
            rows2.at[0], acc_sh.at[pl.ds(s * ROWS_PER_TILE + k * CHUNK,
                                         CHUNK)])
    plsc.subcore_barrier()

    for half in range(2):
        row0 = g * N_CHUNKS + half * HALF_CH
        pltpu.sync_copy(src_hbm.at[pl.ds(row0, HALF_CH)], idx_s)
        pltpu.sync_copy(dst_hbm.at[pl.ds(row0, HALF_CH)], idx_d)
        pltpu.sync_copy(w_hbm.at[pl.ds(row0, HALF_CH)], w_all)

        pltpu.async_copy(y_hbm.at[idx_s.at[0]], rows2.at[0], sem_g)

        def _chunk(j, _):
            p = lax.rem(j, 2)
            q = 1 - p
            pltpu.make_async_copy(y_hbm.at[idx_s.at[j]], rows2.at[p],
                                  sem_g).wait()

            @pl.when(j >= 1)
            def _():
                pltpu.make_async_copy(rows2.at[q],
                                      acc_sh.at[idx_d.at[j - 1]],
                                      sem_s).wait()

            @pl.when(j <= HALF_CH - 2)
            def _():
                pltpu.async_copy(y_hbm.at[idx_s.at[j + 1]], rows2.at[q],
                                 sem_g)

            @plsc.parallel_loop(0, CHUNK, step=1, unroll=4)
            def _scale(r):
                wspl = plsc.load_gather(
                    w_all, [jnp.full((L,), j, jnp.int32),
                            jnp.full((L,), r, jnp.int32)])
                for col in range(D // L):
                    sl = pl.ds(col * L, L)
                    rows2[p, r, sl] = rows2[p, r, sl] * wspl

            pltpu.async_copy(rows2.at[p], acc_sh.at[idx_d.at[j]], sem_s,
                             add=True)
            return 0
        lax.fori_loop(0, HALF_CH, _chunk, 0)
        pltpu.make_async_copy(rows2.at[(HALF_CH - 1) % 2],
                              acc_sh.at[idx_d.at[HALF_CH - 1]],
                              sem_s).wait()
    plsc.subcore_barrier()

    pltpu.sync_copy(acc_sh.at[pl.ds(s * ROWS_PER_TILE, ROWS_PER_TILE)],
                    out_hbm.at[c, pl.ds(s * ROWS_PER_TILE, ROWS_PER_TILE)])


def _mm_body(x_ref, w0_ref, degp_ref, y_ref):
    deg = degp_ref[:, 0] + degp_ref[:, 1] + 1.0
    dinv = jnp.where(deg > 0, lax.rsqrt(deg), 0.0)
    xw = lax.dot_general(x_ref[...], w0_ref[...], (((1,), (1,)), ((), ())),
                         preferred_element_type=jnp.float32)
    y_ref[...] = dinv[:, None] * xw


_MM_BLK = 1000


def _matmul_y(X, W0, degp_t):
    grid = N // _MM_BLK
    return pl.pallas_call(
        _mm_body,
        grid=(grid,),
        in_specs=[
            pl.BlockSpec((_MM_BLK, D), lambda i: (i, 0)),
            pl.BlockSpec((D, D), lambda i: (0, 0)),
            pl.BlockSpec((_MM_BLK, NC), lambda i: (i, 0)),
        ],
        out_specs=pl.BlockSpec((_MM_BLK, D), lambda i: (i, 0)),
        out_shape=jax.ShapeDtypeStruct((N, D), jnp.float32),
    )(X, W0, degp_t)


def _final_body(accp_ref, y_ref, degp_ref, b0_ref, g0_ref, be0_ref,
                w1_ref, b1_ref, out_ref):
    deg = degp_ref[0, :N] + degp_ref[1, :N] + 1.0
    dinv = jnp.where(deg > 0, lax.rsqrt(deg), 0.0)
    acc = accp_ref[0, :N, :] + accp_ref[1, :N, :]
    h = dinv[:, None] * (acc + y_ref[...]) + b0_ref[...]
    mean = jnp.mean(h, axis=0)
    var = jnp.mean((h - mean) ** 2, axis=0)
    h = (h - mean) / jnp.sqrt(var + 1e-5) * g0_ref[...] + be0_ref[...]
    h = jnp.where(h >= 0, h, 0.01 * h)
    out_ref[...] = lax.dot_general(
        h, w1_ref[...], (((1,), (1,)), ((), ())),
        preferred_element_type=jnp.float32) + b1_ref[...]


def _final(accp, y, degp, b0, gamma0, beta0, W1, b1):
    return pl.pallas_call(
        _final_body,
        out_shape=jax.ShapeDtypeStruct((N, D), jnp.float32),
    )(accp, y, degp, b0, gamma0, beta0, W1, b1)


def kernel(X, A, W, W0, b0, gamma0, beta0, W1, b1):
    pad = E_PAD - E
    pad_idx = jnp.arange(pad, dtype=A.dtype) % N
    src = jnp.concatenate([A[0], pad_idx])
    dst = jnp.concatenate([A[1], pad_idx])
    w = jnp.concatenate([W, jnp.zeros((pad,), W.dtype)])
    src2 = src.reshape(NCH_TOT, CHUNK)
    dst2 = dst.reshape(NCH_TOT, CHUNK)
    w2 = w.reshape(NCH_TOT, CHUNK)

    degp = _deg_kernel(dst2, w2)
    y = _matmul_y(X, W0, degp.T)
    accp = _agg_kernel(y, src2, dst2, w2)
    return _final(accp, y, degp, b0, gamma0, beta0, W1, b1)


# speedup vs baseline: 37.5872x; 1.0005x over previous
import functools

import jax
import jax.numpy as jnp
from jax import lax
from jax.experimental import pallas as pl
from jax.experimental.pallas import tpu as pltpu
from jax.experimental.pallas import tpu_sc as plsc

N = 10000
E = 320000
D = 128

NC = 2
NS = 16
L = 16
NW = NC * NS

N_PAD = 10240
ROWS_PER_TILE = N_PAD // NS
CHUNK = 128
PER_W = 10240
E_PAD = PER_W * NW
N_CHUNKS = PER_W // CHUNK
HALF_CH = N_CHUNKS // 2
NCH_TOT = E_PAD // CHUNK

_mesh = plsc.VectorSubcoreMesh(core_axis_name="c", subcore_axis_name="s")


@functools.partial(
    pl.kernel,
    out_type=jax.ShapeDtypeStruct((NC, N_PAD), jnp.float32),
    mesh=_mesh,
    scratch_types=[
        pltpu.VMEM((N_CHUNKS, CHUNK), jnp.int32),
        pltpu.VMEM((N_CHUNKS, CHUNK), jnp.float32),
        pltpu.VMEM((ROWS_PER_TILE,), jnp.float32),
        pltpu.VMEM_SHARED((N_PAD,), jnp.float32),
        pltpu.SemaphoreType.DMA,
    ],
)
def _deg_kernel(dst_hbm, w_hbm, out_hbm, idx_all, w_all, zbuf, deg_sh, sem):
    c = lax.axis_index("c")
    s = lax.axis_index("s")
    g = c * NS + s

    def _z(i, _):
        zbuf[pl.ds(i * L, L)] = jnp.zeros((L,), jnp.float32)
        return 0
    lax.fori_loop(0, ROWS_PER_TILE // L, _z, 0)
    pltpu.sync_copy(zbuf, deg_sh.at[pl.ds(s * ROWS_PER_TILE, ROWS_PER_TILE)])
    plsc.subcore_barrier()

    row0 = g * N_CHUNKS
    pltpu.sync_copy(dst_hbm.at[pl.ds(row0, N_CHUNKS)], idx_all)
    pltpu.sync_copy(w_hbm.at[pl.ds(row0, N_CHUNKS)], w_all)

    K = 16
    for j0 in range(0, N_CHUNKS, K):
        jn = min(j0 + K, N_CHUNKS)
        for j in range(j0, jn):
            pltpu.async_copy(w_all.at[j], deg_sh.at[idx_all.at[j]], sem,
                             add=True)
        for j in range(j0, jn):
            pltpu.make_async_copy(w_all.at[j], deg_sh.at[idx_all.at[j]],
                                  sem).wait()
    plsc.subcore_barrier()

    pltpu.sync_copy(deg_sh.at[pl.ds(s * ROWS_PER_TILE, ROWS_PER_TILE)],
                    out_hbm.at[c, pl.ds(s * ROWS_PER_TILE, ROWS_PER_TILE)])


@functools.partial(
    pl.kernel,
    out_type=jax.ShapeDtypeStruct((NC, N_PAD, D), jnp.float32),
    mesh=_mesh,
    scratch_types=[
        pltpu.VMEM((HALF_CH, CHUNK), jnp.int32),
        pltpu.VMEM((HALF_CH, CHUNK), jnp.int32),
        pltpu.VMEM((HALF_CH, CHUNK), jnp.float32),
        pltpu.VMEM((2, CHUNK, D), jnp.float32),
        pltpu.VMEM_SHARED((N_PAD, D), jnp.float32),
        pltpu.SemaphoreType.DMA,
        pltpu.SemaphoreType.DMA,
    ],
    compiler_params=pltpu.CompilerParams(needs_layout_passes=False),
)
def _agg_kernel(y_hbm, src_hbm, dst_hbm, w_hbm, out_hbm,
                idx_s, idx_d, w_all, rows32, acc_sh, sem_g, sem_s):
    c = lax.axis_index("c")
    s = lax.axis_index("s")
    g = c * NS + s

    def _zr(r, _):
        for col in range(D // L):
            rows32[0, r, pl.ds(col * L, L)] = jnp.zeros((L,), jnp.float32)
        return 0
    lax.fori_loop(0, CHUNK, _zr, 0)
    for k in range(ROWS_PER_TILE // CHUNK):
        pltpu.sync_copy(
            rows32.at[0],
            acc_sh.at[pl.ds(s * ROWS_PER_TILE + k * CHUNK, CHUNK)])
    plsc.subcore_barrier()

    for half in range(2):
        row0 = g * N_CHUNKS + half * HALF_CH
        pltpu.sync_copy(src_hbm.at[pl.ds(row0, HALF_CH)], idx_s)
        pltpu.sync_copy(dst_hbm.at[pl.ds(row0, HALF_CH)], idx_d)
        pltpu.sync_copy(w_hbm.at[pl.ds(row0, HALF_CH)], w_all)

        pltpu.async_copy(y_hbm.at[idx_s.at[0]], rows32.at[0], sem_g)

        def _step(j, p):
            q = 1 - p
            pltpu.make_async_copy(y_hbm.at[idx_s.at[j]], rows32.at[p],
                                  sem_g).wait()

            @pl.when(j >= 1)
            def _():
                pltpu.make_async_copy(rows32.at[q],
                                      acc_sh.at[idx_d.at[j - 1]],
                                      sem_s).wait()

            @pl.when(j <= HALF_CH - 2)
            def _():
                pltpu.async_copy(y_hbm.at[idx_s.at[j + 1]], rows32.at[q],
                                 sem_g)

            @plsc.parallel_loop(0, CHUNK, step=1, unroll=4)
            def _scale(r):
                wspl = plsc.load_gather(
                    w_all, [jnp.full((L,), j, jnp.int32),
                            jnp.full((L,), r, jnp.int32)])
                for col in range(D // L):
                    sl = pl.ds(col * L, L)
                    rows32[p, r, sl] = rows32[p, r, sl] * wspl

            pltpu.async_copy(rows32.at[p], acc_sh.at[idx_d.at[j]], sem_s,
                             add=True)

        def _chunk2(i, _):
            _step(2 * i, 0)
            _step(2 * i + 1, 1)
            return 0
        lax.fori_loop(0, HALF_CH // 2, _chunk2, 0)
        pltpu.make_async_copy(rows32.at[(HALF_CH - 1) % 2],
                              acc_sh.at[idx_d.at[HALF_CH - 1]],
                              sem_s).wait()
    plsc.subcore_barrier()

    pltpu.sync_copy(acc_sh.at[pl.ds(s * ROWS_PER_TILE, ROWS_PER_TILE)],
                    out_hbm.at[c, pl.ds(s * ROWS_PER_TILE, ROWS_PER_TILE)])


def _mm_body(x_ref, w0_ref, degp_ref, y_ref):
    deg = degp_ref[:, 0] + degp_ref[:, 1] + 1.0
    dinv = jnp.where(deg > 0, lax.rsqrt(deg), 0.0)
    xw = lax.dot_general(x_ref[...], w0_ref[...], (((1,), (1,)), ((), ())),
                         preferred_element_type=jnp.float32)
    y_ref[...] = dinv[:, None] * xw


_MM_BLK = 1000


def _matmul_y(X, W0, degp_t):
    grid = N // _MM_BLK
    return pl.pallas_call(
        _mm_body,
        grid=(grid,),
        in_specs=[
            pl.BlockSpec((_MM_BLK, D), lambda i: (i, 0)),
            pl.BlockSpec((D, D), lambda i: (0, 0)),
            pl.BlockSpec((_MM_BLK, NC), lambda i: (i, 0)),
        ],
        out_specs=pl.BlockSpec((_MM_BLK, D), lambda i: (i, 0)),
        out_shape=jax.ShapeDtypeStruct((N, D), jnp.float32),
    )(X, W0, degp_t)


def _final_body(accp_ref, y_ref, degp_ref, b0_ref, g0_ref, be0_ref,
                w1_ref, b1_ref, out_ref):
    deg = degp_ref[0, :N] + degp_ref[1, :N] + 1.0
    dinv = jnp.where(deg > 0, lax.rsqrt(deg), 0.0)
    acc = accp_ref[0, :N, :] + accp_ref[1, :N, :]
    h = dinv[:, None] * (acc + y_ref[...]) + b0_ref[...]
    mean = jnp.mean(h, axis=0)
    var = jnp.mean((h - mean) ** 2, axis=0)
    h = (h - mean) / jnp.sqrt(var + 1e-5) * g0_ref[...] + be0_ref[...]
    h = jnp.where(h >= 0, h, 0.01 * h)
    out_ref[...] = lax.dot_general(
        h, w1_ref[...], (((1,), (1,)), ((), ())),
        preferred_element_type=jnp.float32) + b1_ref[...]


def _final(accp, y, degp, b0, gamma0, beta0, W1, b1):
    return pl.pallas_call(
        _final_body,
        out_shape=jax.ShapeDtypeStruct((N, D), jnp.float32),
    )(accp, y, degp, b0, gamma0, beta0, W1, b1)


def kernel(X, A, W, W0, b0, gamma0, beta0, W1, b1):
    pad = E_PAD - E
    pad_idx = jnp.arange(pad, dtype=A.dtype) % N
    src = jnp.concatenate([A[0], pad_idx])
    dst = jnp.concatenate([A[1], pad_idx])
    w = jnp.concatenate([W, jnp.zeros((pad,), W.dtype)])
    src2 = src.reshape(NCH_TOT, CHUNK)
    dst2 = dst.reshape(NCH_TOT, CHUNK)
    w2 = w.reshape(NCH_TOT, CHUNK)

    degp = _deg_kernel(dst2, w2)
    y = _matmul_y(X, W0, degp.T)
    accp = _agg_kernel(y, src2, dst2, w2)
    return _final(accp, y, degp, b0, gamma0, beta0, W1, b1)
